# baseline (speedup 1.0000x reference)
import math
import functools

import jax
import jax.numpy as jnp
from jax import lax
from jax.experimental import pallas as pl
from jax.experimental.pallas import tpu as pltpu
from jax.experimental.pallas import tpu_sc as plsc

D = 32
L = 16
NC = 2
NS = 16
NW = NC * NS
SCALE = math.sqrt(D)

H = 50
B = 16384
BW = B // NW
NT = BW // 128
DT = D // 8

_mesh = plsc.VectorSubcoreMesh(core_axis_name="c", subcore_axis_name="s")

VOCAB = 1000000


@functools.partial(
    pl.kernel,
    out_type=jax.ShapeDtypeStruct((H, DT, B // 128, 8, 128),
                                  jnp.float32),
    mesh=_mesh,
    scratch_types=[
        pltpu.VMEM((H, BW), jnp.int32),
        pltpu.VMEM((BW, D), jnp.float32),
        pltpu.VMEM((BW, D), jnp.float32),
        pltpu.VMEM((DT, NT, 8, 128), jnp.float32),
        pltpu.VMEM((DT, NT, 8, 128), jnp.float32),
        pltpu.SemaphoreType.DMA,
        pltpu.SemaphoreType.DMA,
        pltpu.SemaphoreType.DMA,
        pltpu.SemaphoreType.DMA,
    ],
    compiler_params=pltpu.CompilerParams(
        use_tc_tiling_on_sc=False, needs_layout_passes=False),
)
def _emb_kernel(xt_hbm, table_hbm, out_hbm, idx_all, rows0, rows1,
                tbuf0, tbuf1, gsem0, gsem1, osem0, osem1):
    wid = lax.axis_index("s") * NC + lax.axis_index("c")
    b0 = wid * BW
    bt0 = wid * NT

    rows = (rows0, rows1)
    tbufs = (tbuf0, tbuf1)
    gsems = (gsem0, gsem1)
    osems = (osem0, osem1)

    pltpu.sync_copy(xt_hbm.at[:, pl.ds(b0, BW)], idx_all)

    def issue_gather(h, buf):
        pltpu.async_copy(
            table_hbm.at[idx_all.at[h]], rows[buf], gsems[buf])

    def wait_gather(h, buf):
        pltpu.make_async_copy(
            table_hbm.at[idx_all.at[h]], rows[buf], gsems[buf]).wait()

    def wait_out(h, buf):
        pltpu.make_async_copy(
            tbufs[buf], out_hbm.at[h, :, pl.ds(bt0, NT)], osems[buf]).wait()

    iota16 = jax.lax.iota(jnp.int32, L)

    def compute(h, buf):
        r = rows[buf]
        tb = tbufs[buf]

        @plsc.parallel_loop(0, BW // L, unroll=4)
        def _(i2):
            lvec = i2 * L + iota16
            t = i2 // 8
            j16 = (i2 % 8) * L
            for d in range(D):
                dt, s = divmod(d, 8)
                g = plsc.load_gather(r, [lvec, jnp.full((L,), d, jnp.int32)])
                tb[dt, t, s, pl.ds(j16, L)] = g * SCALE

        pltpu.async_copy(tb, out_hbm.at[h, :, pl.ds(bt0, NT)], osems[buf])

    issue_gather(0, 0)
    issue_gather(1, 1)

    def step(i, _):
        h0 = 2 * i
        h1 = 2 * i + 1

        wait_gather(h0, 0)

        @pl.when(i >= 1)
        def _():
            wait_out(h0, 0)

        compute(h0, 0)

        @pl.when(h0 + 2 < H)
        def _():
            issue_gather(h0 + 2, 0)

        wait_gather(h1, 1)

        @pl.when(i >= 1)
        def _():
            wait_out(h1, 1)

        compute(h1, 1)

        @pl.when(h1 + 2 < H)
        def _():
            issue_gather(h1 + 2, 1)

        return 0

    lax.fori_loop(0, H // 2, step, 0)
    wait_out(H - 2, 0)
    wait_out(H - 1, 1)


def kernel(X, table):
    xt = X.T.astype(jnp.int32)
    outp = _emb_kernel(xt, table)
    out = outp.transpose(0, 1, 3, 2, 4).reshape(H, D, B).transpose(2, 0, 1)
    return out


# speedup vs baseline: 1.0928x; 1.0928x over previous
import math
import functools

import jax
import jax.numpy as jnp
from jax import lax
from jax.experimental import pallas as pl
from jax.experimental.pallas import tpu as pltpu
from jax.experimental.pallas import tpu_sc as plsc

D = 32
L = 16
NC = 2
NS = 16
NW = NC * NS
SCALE = math.sqrt(D)

H = 50
B = 16384
BW = B // NW
NT = BW // 128
DT = D // 8

_mesh = plsc.VectorSubcoreMesh(core_axis_name="c", subcore_axis_name="s")

VOCAB = 1000000


@functools.partial(
    pl.kernel,
    out_type=jax.ShapeDtypeStruct((H, DT, B // 128, 8, 128),
                                  jnp.float32),
    mesh=_mesh,
    scratch_types=[
        pltpu.VMEM((H, BW), jnp.int32),
        pltpu.VMEM((BW, D), jnp.float32),
        pltpu.VMEM((BW, D), jnp.float32),
        pltpu.VMEM((DT, NT, 8, 128), jnp.float32),
        pltpu.VMEM((DT, NT, 8, 128), jnp.float32),
        pltpu.SemaphoreType.DMA,
        pltpu.SemaphoreType.DMA,
        pltpu.SemaphoreType.DMA,
        pltpu.SemaphoreType.DMA,
    ],
    compiler_params=pltpu.CompilerParams(
        use_tc_tiling_on_sc=False, needs_layout_passes=False),
)
def _emb_kernel(xt_hbm, table_hbm, out_hbm, idx_all, rows0, rows1,
                tbuf0, tbuf1, gsem0, gsem1, osem0, osem1):
    wid = lax.axis_index("s") * NC + lax.axis_index("c")
    b0 = wid * BW
    bt0 = wid * NT

    rows = (rows0, rows1)
    tbufs = (tbuf0, tbuf1)
    gsems = (gsem0, gsem1)
    osems = (osem0, osem1)

    pltpu.sync_copy(xt_hbm.at[:, pl.ds(b0, BW)], idx_all)

    def issue_gather(h, buf):
        pltpu.async_copy(
            table_hbm.at[idx_all.at[h]], rows[buf], gsems[buf])

    def wait_gather(h, buf):
        pltpu.make_async_copy(
            table_hbm.at[idx_all.at[h]], rows[buf], gsems[buf]).wait()

    def wait_out(h, buf):
        pltpu.make_async_copy(
            tbufs[buf], out_hbm.at[h, :, pl.ds(bt0, NT)], osems[buf]).wait()

    iota16 = jax.lax.iota(jnp.int32, L)

    def compute(h, buf):
        r = rows[buf]
        tb = tbufs[buf]

        @plsc.parallel_loop(0, (BW // L) * DT, unroll=2)
        def _(k):
            i2 = k // DT
            dg = k % DT
            lvec = i2 * L + iota16
            t = i2 // 8
            j16 = (i2 % 8) * L
            for di in range(8):
                dsplat = lax.broadcast(dg * 8 + di, (L,))
                g = plsc.load_gather(r, [lvec, dsplat])
                tb[dg, t, di, pl.ds(j16, L)] = g * SCALE

        pltpu.async_copy(tb, out_hbm.at[h, :, pl.ds(bt0, NT)], osems[buf])

    issue_gather(0, 0)
    issue_gather(1, 1)

    def step(i, _):
        h0 = 2 * i
        h1 = 2 * i + 1

        wait_gather(h0, 0)

        @pl.when(i >= 1)
        def _():
            wait_out(h0, 0)

        compute(h0, 0)

        @pl.when(h0 + 2 < H)
        def _():
            issue_gather(h0 + 2, 0)

        wait_gather(h1, 1)

        @pl.when(i >= 1)
        def _():
            wait_out(h1, 1)

        compute(h1, 1)

        @pl.when(h1 + 2 < H)
        def _():
            issue_gather(h1 + 2, 1)

        return 0

    lax.fori_loop(0, H // 2, step, 0)
    wait_out(H - 2, 0)
    wait_out(H - 1, 1)


def kernel(X, table):
    xt = X.T.astype(jnp.int32)
    outp = _emb_kernel(xt, table)
    out = outp.transpose(0, 1, 3, 2, 4).reshape(H, D, B).transpose(2, 0, 1)
    return out


# speedup vs baseline: 1.0934x; 1.0005x over previous
import math
import functools

import jax
import jax.numpy as jnp
from jax import lax
from jax.experimental import pallas as pl
from jax.experimental.pallas import tpu as pltpu
from jax.experimental.pallas import tpu_sc as plsc

D = 32
L = 16
NC = 2
NS = 16
NW = NC * NS
SCALE = math.sqrt(D)

H = 50
B = 16384
BW = B // NW
NT = BW // 128
DT = D // 8

_mesh = plsc.VectorSubcoreMesh(core_axis_name="c", subcore_axis_name="s")

VOCAB = 1000000


@functools.partial(
    pl.kernel,
    out_type=jax.ShapeDtypeStruct((H, DT, B // 128, 8, 128),
                                  jnp.float32),
    mesh=_mesh,
    scratch_types=[
        pltpu.VMEM((H, BW), jnp.int32),
        pltpu.VMEM((BW, D), jnp.float32),
        pltpu.VMEM((BW, D), jnp.float32),
        pltpu.VMEM((DT, NT, 8, 128), jnp.float32),
        pltpu.VMEM((DT, NT, 8, 128), jnp.float32),
        pltpu.SemaphoreType.DMA,
        pltpu.SemaphoreType.DMA,
        pltpu.SemaphoreType.DMA,
        pltpu.SemaphoreType.DMA,
    ],
    compiler_params=pltpu.CompilerParams(
        use_tc_tiling_on_sc=False, needs_layout_passes=False),
)
def _emb_kernel(xt_hbm, table_hbm, out_hbm, idx_all, rows0, rows1,
                tbuf0, tbuf1, gsem0, gsem1, osem0, osem1):
    wid = lax.axis_index("s") * NC + lax.axis_index("c")
    b0 = wid * BW
    bt0 = wid * NT

    rows = (rows0, rows1)
    tbufs = (tbuf0, tbuf1)
    gsems = (gsem0, gsem1)
    osems = (osem0, osem1)

    pltpu.sync_copy(xt_hbm.at[:, pl.ds(b0, BW)], idx_all)

    def issue_gather(h, buf):
        pltpu.async_copy(
            table_hbm.at[idx_all.at[h]], rows[buf], gsems[buf])

    def wait_gather(h, buf):
        pltpu.make_async_copy(
            table_hbm.at[idx_all.at[h]], rows[buf], gsems[buf]).wait()

    def wait_out(h, buf):
        pltpu.make_async_copy(
            tbufs[buf], out_hbm.at[h, :, pl.ds(bt0, NT)], osems[buf]).wait()

    iota16 = jax.lax.iota(jnp.int32, L)

    def compute(h, buf):
        r = rows[buf]
        tb = tbufs[buf]

        @plsc.parallel_loop(0, (BW // L) * DT, unroll=2)
        def _(k):
            i2 = k // DT
            dg = k % DT
            lvec = i2 * L + iota16
            t = i2 // 8
            j16 = (i2 % 8) * L
            for di in range(8):
                dsplat = lax.broadcast(dg * 8 + di, (L,))
                g = plsc.load_gather(r, [lvec, dsplat])
                tb[dg, t, di, pl.ds(j16, L)] = g * SCALE

        pltpu.async_copy(tb, out_hbm.at[h, :, pl.ds(bt0, NT)], osems[buf])

    issue_gather(0, 0)
    issue_gather(1, 1)

    def step(i, _):
        h0 = 2 * i
        h1 = 2 * i + 1

        wait_gather(h0, 0)

        @pl.when(i >= 1)
        def _():
            wait_out(h0, 0)

        compute(h0, 0)

        @pl.when(h0 + 2 < H)
        def _():
            issue_gather(h0 + 2, 0)

        wait_gather(h1, 1)

        @pl.when(i >= 1)
        def _():
            wait_out(h1, 1)

        compute(h1, 1)

        @pl.when(h1 + 2 < H)
        def _():
            issue_gather(h1 + 2, 1)

        return 0

    lax.fori_loop(0, H // 2, step, 0)
    wait_out(H - 2, 0)
    wait_out(H - 1, 1)


def kernel(X, table):
    xt = X.T.astype(jnp.int32)
    table3 = jax.lax.optimization_barrier(
        table.reshape(VOCAB * D // 1024, 8, 128))
    table_lin = table3.reshape(VOCAB, D)
    outp = _emb_kernel(xt, table_lin)
    out = outp.transpose(0, 1, 3, 2, 4).reshape(H, D, B).transpose(2, 0, 1)
    return out


# speedup vs baseline: 1.0949x; 1.0014x over previous
import math
import functools

import jax
import jax.numpy as jnp
from jax import lax
from jax.experimental import pallas as pl
from jax.experimental.pallas import tpu as pltpu
from jax.experimental.pallas import tpu_sc as plsc

D = 32
L = 16
NC = 2
NS = 16
NW = NC * NS
SCALE = math.sqrt(D)

H = 50
B = 16384
BW = B // NW
NT = BW // 128
DT = D // 8

_mesh = plsc.VectorSubcoreMesh(core_axis_name="c", subcore_axis_name="s")

VOCAB = 1000000


@functools.partial(
    pl.kernel,
    out_type=jax.ShapeDtypeStruct((H, DT, B // 128, 8, 128),
                                  jnp.float32),
    mesh=_mesh,
    scratch_types=[
        pltpu.VMEM((H, BW), jnp.int32),
        pltpu.VMEM((BW, D), jnp.float32),
        pltpu.VMEM((BW, D), jnp.float32),
        pltpu.VMEM((DT, NT, 8, 128), jnp.float32),
        pltpu.VMEM((DT, NT, 8, 128), jnp.float32),
        pltpu.SemaphoreType.DMA,
        pltpu.SemaphoreType.DMA,
        pltpu.SemaphoreType.DMA,
        pltpu.SemaphoreType.DMA,
    ],
    compiler_params=pltpu.CompilerParams(
        use_tc_tiling_on_sc=False, needs_layout_passes=False),
)
def _emb_kernel(xt_hbm, table_hbm, out_hbm, idx_all, rows0, rows1,
                tbuf0, tbuf1, gsem0, gsem1, osem0, osem1):
    wid = lax.axis_index("s") * NC + lax.axis_index("c")
    b0 = wid * BW
    bt0 = wid * NT

    rows = (rows0, rows1)
    tbufs = (tbuf0, tbuf1)
    gsems = (gsem0, gsem1)
    osems = (osem0, osem1)

    pltpu.sync_copy(xt_hbm.at[:, pl.ds(b0, BW)], idx_all)

    def issue_gather(h, buf):
        pltpu.async_copy(
            table_hbm.at[idx_all.at[h]], rows[buf], gsems[buf])

    def wait_gather(h, buf):
        pltpu.make_async_copy(
            table_hbm.at[idx_all.at[h]], rows[buf], gsems[buf]).wait()

    def wait_out(h, buf):
        pltpu.make_async_copy(
            tbufs[buf], out_hbm.at[h, :, pl.ds(bt0, NT)], osems[buf]).wait()

    iota16 = jax.lax.iota(jnp.int32, L)

    def compute(h, buf):
        r = rows[buf]
        tb = tbufs[buf]

        @plsc.parallel_loop(0, (BW // L) * DT, unroll=4)
        def _(k):
            i2 = k // DT
            dg = k % DT
            lvec = i2 * L + iota16
            t = i2 // 8
            j16 = (i2 % 8) * L
            for di in range(8):
                dsplat = lax.broadcast(dg * 8 + di, (L,))
                g = plsc.load_gather(r, [lvec, dsplat])
                tb[dg, t, di, pl.ds(j16, L)] = g * SCALE

        pltpu.async_copy(tb, out_hbm.at[h, :, pl.ds(bt0, NT)], osems[buf])

    issue_gather(0, 0)
    issue_gather(1, 1)

    def step(i, _):
        h0 = 2 * i
        h1 = 2 * i + 1

        wait_gather(h0, 0)

        @pl.when(i >= 1)
        def _():
            wait_out(h0, 0)

        compute(h0, 0)

        @pl.when(h0 + 2 < H)
        def _():
            issue_gather(h0 + 2, 0)

        wait_gather(h1, 1)

        @pl.when(i >= 1)
        def _():
            wait_out(h1, 1)

        compute(h1, 1)

        @pl.when(h1 + 2 < H)
        def _():
            issue_gather(h1 + 2, 1)

        return 0

    lax.fori_loop(0, H // 2, step, 0)
    wait_out(H - 2, 0)
    wait_out(H - 1, 1)


def kernel(X, table):
    xt = X.T.astype(jnp.int32)
    table3 = jax.lax.optimization_barrier(
        table.reshape(VOCAB * D // 1024, 8, 128))
    table_lin = table3.reshape(VOCAB, D)
    outp = _emb_kernel(xt, table_lin)
    out = outp.transpose(0, 1, 3, 2, 4).reshape(H, D, B).transpose(2, 0, 1)
    return out
